# baseline (speedup 1.0000x reference)
import functools

import jax
import jax.numpy as jnp
from jax import lax
from jax.experimental import pallas as pl
from jax.experimental.pallas import tpu as pltpu
from jax.experimental.pallas import tpu_sc as plsc

VOCAB = 100000
EMBED_DIM = 32
HIDDEN = 128
BATCH = 1024
CTX = 200

_L = 16


_C0 = 128
_C1 = CTX - _C0


def _sc_pool_kernel(
    emb_hbm, idx_hbm, out_hbm, idx0_v, idx1_v, rows0_v, rows1_v, pooled_v, sem
):
    nc = 2
    b_per_w = BATCH // 32
    wid = lax.axis_index("s") * nc + lax.axis_index("c")
    base = wid * b_per_w

    inv = jnp.full((_L,), 1.0 / CTX, dtype=jnp.float32)

    def row_body(i, _):
        row = base + i
        pltpu.sync_copy(idx_hbm.at[row, pl.ds(0, _C0)], idx0_v)
        pltpu.sync_copy(idx_hbm.at[row, pl.ds(_C0, _C1)], idx1_v)
        pltpu.async_copy(emb_hbm.at[idx0_v], rows0_v, sem).wait()
        pltpu.async_copy(emb_hbm.at[idx1_v], rows1_v, sem).wait()

        def acc0_body(j, carry):
            a0, a1 = carry
            for u in range(8):
                r = j * 8 + u
                a0 = a0 + rows0_v[r, 0:_L]
                a1 = a1 + rows0_v[r, _L : 2 * _L]
            return (a0, a1)

        def acc1_body(j, carry):
            a0, a1 = carry
            for u in range(8):
                r = j * 8 + u
                a0 = a0 + rows1_v[r, 0:_L]
                a1 = a1 + rows1_v[r, _L : 2 * _L]
            return (a0, a1)

        z = jnp.zeros((_L,), dtype=jnp.float32)
        a0, a1 = lax.fori_loop(0, _C0 // 8, acc0_body, (z, z))
        a0, a1 = lax.fori_loop(0, _C1 // 8, acc1_body, (a0, a1))
        pooled_v[i, 0:_L] = a0 * inv
        pooled_v[i, _L : 2 * _L] = a1 * inv
        return 0

    lax.fori_loop(0, b_per_w, row_body, 0)
    pltpu.sync_copy(pooled_v, out_hbm.at[pl.ds(base, b_per_w)])


def _sc_pool(emb, idx):
    b_per_w = BATCH // 32
    mesh = plsc.VectorSubcoreMesh(core_axis_name="c", subcore_axis_name="s")
    return pl.kernel(
        _sc_pool_kernel,
        mesh=mesh,
        out_type=jax.ShapeDtypeStruct((BATCH, EMBED_DIM), jnp.float32),
        scratch_types=[
            pltpu.VMEM((_C0,), jnp.int32),
            pltpu.VMEM((_C1,), jnp.int32),
            pltpu.VMEM((_C0, EMBED_DIM), jnp.float32),
            pltpu.VMEM((_C1, EMBED_DIM), jnp.float32),
            pltpu.VMEM((b_per_w, EMBED_DIM), jnp.float32),
            pltpu.SemaphoreType.DMA,
        ],
        compiler_params=pltpu.CompilerParams(use_tc_tiling_on_sc=False),
    )(emb, idx)


_TW = 2048
_NT = pl.cdiv(VOCAB, _TW)
_TAIL = VOCAB - (_NT - 1) * _TW
_NBUF = 5


def _mlp_block(
    pooled_ref, w1_ref, b1_ref, w2_ref, b2_ref, out_hbm, h_ref, obuf, tailbuf, sems
):
    j = pl.program_id(0)

    @pl.when(j == 0)
    def _():
        h = (
            jnp.dot(pooled_ref[...], w1_ref[...], preferred_element_type=jnp.float32)
            + b1_ref[...]
        )
        h_ref[...] = jnp.maximum(h, 0.0)

    slot = lax.rem(j, _NBUF)

    @pl.when(j >= _NBUF)
    def _():
        pltpu.make_async_copy(
            obuf.at[slot],
            out_hbm.at[:, pl.ds((j - _NBUF) * _TW, _TW)],
            sems.at[slot],
        ).wait()

    blk = (
        jnp.dot(h_ref[...], w2_ref[...], preferred_element_type=jnp.float32)
        + b2_ref[...]
    )

    @pl.when(j < _NT - 1)
    def _():
        obuf[slot] = blk
        pltpu.make_async_copy(
            obuf.at[slot], out_hbm.at[:, pl.ds(j * _TW, _TW)], sems.at[slot]
        ).start()

    @pl.when(j == _NT - 1)
    def _():
        tailbuf[...] = blk[:, :_TAIL]
        pltpu.make_async_copy(
            tailbuf, out_hbm.at[:, pl.ds((_NT - 1) * _TW, _TAIL)], sems.at[slot]
        ).start()
        for step in range(_NT - _NBUF, _NT):
            s = step % _NBUF
            if step == _NT - 1:
                pltpu.make_async_copy(
                    tailbuf, out_hbm.at[:, pl.ds(step * _TW, _TAIL)], sems.at[s]
                ).wait()
            else:
                pltpu.make_async_copy(
                    obuf.at[s], out_hbm.at[:, pl.ds(step * _TW, _TW)], sems.at[s]
                ).wait()


def _tc_mlp(pooled, W1, b1, W2, b2):
    grid = (_NT,)
    return pl.pallas_call(
        _mlp_block,
        grid=grid,
        in_specs=[
            pl.BlockSpec((BATCH, EMBED_DIM), lambda j: (0, 0)),
            pl.BlockSpec((EMBED_DIM, HIDDEN), lambda j: (0, 0)),
            pl.BlockSpec((1, HIDDEN), lambda j: (0, 0)),
            pl.BlockSpec((HIDDEN, _TW), lambda j: (0, j)),
            pl.BlockSpec((1, _TW), lambda j: (0, j)),
        ],
        out_specs=pl.BlockSpec(memory_space=pltpu.MemorySpace.HBM),
        out_shape=jax.ShapeDtypeStruct((BATCH, VOCAB), jnp.float32),
        scratch_shapes=[
            pltpu.VMEM((BATCH, HIDDEN), jnp.float32),
            pltpu.VMEM((_NBUF, BATCH, _TW), jnp.float32),
            pltpu.VMEM((BATCH, _TAIL), jnp.float32),
            pltpu.SemaphoreType.DMA((_NBUF,)),
        ],
        compiler_params=pltpu.CompilerParams(
            dimension_semantics=("arbitrary",),
        ),
    )(pooled, W1, b1.reshape(1, HIDDEN), W2, b2.reshape(1, VOCAB))


def kernel(inputs, emb, W1, b1, W2, b2):
    pooled = _sc_pool(emb, inputs.astype(jnp.int32))
    return _tc_mlp(pooled, W1, b1, W2, b2)


# speedup vs baseline: 1.0806x; 1.0806x over previous
import functools

import jax
import jax.numpy as jnp
from jax import lax
from jax.experimental import pallas as pl
from jax.experimental.pallas import tpu as pltpu
from jax.experimental.pallas import tpu_sc as plsc

VOCAB = 100000
EMBED_DIM = 32
HIDDEN = 128
BATCH = 1024
CTX = 200

_L = 16


_NCH = (32 * CTX) // 128
_B_PER_W = BATCH // 32


def _sc_pool_kernel(emb_hbm, idx_hbm, out_hbm, idx_v, r0, r1, r2, pooled_v, *sems):
    nc = 2
    wid = lax.axis_index("s") * nc + lax.axis_index("c")
    base = wid * _B_PER_W

    inv = jnp.full((_L,), 1.0 / CTX, dtype=jnp.float32)
    rows = (r0, r1, r2)
    isem, g0, g1, g2 = sems
    gsems = (g0, g1, g2)

    pltpu.async_copy(idx_hbm.at[wid], idx_v, isem).wait()

    cps = [None, None, None]

    def start_gather(c):
        cps[c % 3] = pltpu.async_copy(
            emb_hbm.at[idx_v.at[c]], rows[c % 3], gsems[c % 3]
        )

    start_gather(0)
    start_gather(1)

    def acc8(buf, lo, hi, a0, a1):
        def body(j, carry):
            b0, b1 = carry
            for u in range(8):
                r = j * 8 + u
                b0 = b0 + buf[r, 0:_L]
                b1 = b1 + buf[r, _L : 2 * _L]
            return (b0, b1)

        return lax.fori_loop(lo // 8, hi // 8, body, (a0, a1))

    z = jnp.zeros((_L,), dtype=jnp.float32)
    a0, a1 = z, z
    for c in range(_NCH):
        if c + 2 < _NCH:
            start_gather(c + 2)
        cps[c % 3].wait()
        buf = rows[c % 3]
        start = 128 * c
        r = start // CTX
        split = min(128, CTX * (r + 1) - start)
        a0, a1 = acc8(buf, 0, split, a0, a1)
        if split < 128:
            pooled_v[r, 0:_L] = a0 * inv
            pooled_v[r, _L : 2 * _L] = a1 * inv
            a0, a1 = acc8(buf, split, 128, z, z)
        elif (start + 128) % CTX == 0:
            pooled_v[r, 0:_L] = a0 * inv
            pooled_v[r, _L : 2 * _L] = a1 * inv
            a0, a1 = z, z

    pltpu.sync_copy(pooled_v, out_hbm.at[pl.ds(base, _B_PER_W)])


def _sc_pool(emb, idx_chunks):
    mesh = plsc.VectorSubcoreMesh(core_axis_name="c", subcore_axis_name="s")
    return pl.kernel(
        _sc_pool_kernel,
        mesh=mesh,
        out_type=jax.ShapeDtypeStruct((BATCH, EMBED_DIM), jnp.float32),
        scratch_types=[
            pltpu.VMEM((_NCH, 128), jnp.int32),
            pltpu.VMEM((128, EMBED_DIM), jnp.float32),
            pltpu.VMEM((128, EMBED_DIM), jnp.float32),
            pltpu.VMEM((128, EMBED_DIM), jnp.float32),
            pltpu.VMEM((_B_PER_W, EMBED_DIM), jnp.float32),
            pltpu.SemaphoreType.DMA,
            pltpu.SemaphoreType.DMA,
            pltpu.SemaphoreType.DMA,
            pltpu.SemaphoreType.DMA,
        ],
        compiler_params=pltpu.CompilerParams(use_tc_tiling_on_sc=False),
    )(emb, idx_chunks)


_TW = 2048
_NT = pl.cdiv(VOCAB, _TW)
_TAIL = VOCAB - (_NT - 1) * _TW
_NBUF = 5


def _mlp_block(
    pooled_ref, w1_ref, b1_ref, w2_ref, b2_ref, out_hbm, h_ref, obuf, tailbuf, sems
):
    j = pl.program_id(0)

    @pl.when(j == 0)
    def _():
        h = (
            jnp.dot(pooled_ref[...], w1_ref[...], preferred_element_type=jnp.float32)
            + b1_ref[...]
        )
        h_ref[...] = jnp.maximum(h, 0.0)

    slot = lax.rem(j, _NBUF)

    @pl.when(j >= _NBUF)
    def _():
        pltpu.make_async_copy(
            obuf.at[slot],
            out_hbm.at[:, pl.ds((j - _NBUF) * _TW, _TW)],
            sems.at[slot],
        ).wait()

    blk = (
        jnp.dot(h_ref[...], w2_ref[...], preferred_element_type=jnp.float32)
        + b2_ref[...]
    )

    @pl.when(j < _NT - 1)
    def _():
        obuf[slot] = blk
        pltpu.make_async_copy(
            obuf.at[slot], out_hbm.at[:, pl.ds(j * _TW, _TW)], sems.at[slot]
        ).start()

    @pl.when(j == _NT - 1)
    def _():
        tailbuf[...] = blk[:, :_TAIL]
        pltpu.make_async_copy(
            tailbuf, out_hbm.at[:, pl.ds((_NT - 1) * _TW, _TAIL)], sems.at[slot]
        ).start()
        for step in range(_NT - _NBUF, _NT):
            s = step % _NBUF
            if step == _NT - 1:
                pltpu.make_async_copy(
                    tailbuf, out_hbm.at[:, pl.ds(step * _TW, _TAIL)], sems.at[s]
                ).wait()
            else:
                pltpu.make_async_copy(
                    obuf.at[s], out_hbm.at[:, pl.ds(step * _TW, _TW)], sems.at[s]
                ).wait()


def _tc_mlp(pooled, W1, b1, W2, b2):
    grid = (_NT,)
    return pl.pallas_call(
        _mlp_block,
        grid=grid,
        in_specs=[
            pl.BlockSpec((BATCH, EMBED_DIM), lambda j: (0, 0)),
            pl.BlockSpec((EMBED_DIM, HIDDEN), lambda j: (0, 0)),
            pl.BlockSpec((1, HIDDEN), lambda j: (0, 0)),
            pl.BlockSpec((HIDDEN, _TW), lambda j: (0, j)),
            pl.BlockSpec((1, _TW), lambda j: (0, j)),
        ],
        out_specs=pl.BlockSpec(memory_space=pltpu.MemorySpace.HBM),
        out_shape=jax.ShapeDtypeStruct((BATCH, VOCAB), jnp.float32),
        scratch_shapes=[
            pltpu.VMEM((BATCH, HIDDEN), jnp.float32),
            pltpu.VMEM((_NBUF, BATCH, _TW), jnp.float32),
            pltpu.VMEM((BATCH, _TAIL), jnp.float32),
            pltpu.SemaphoreType.DMA((_NBUF,)),
        ],
        compiler_params=pltpu.CompilerParams(
            dimension_semantics=("arbitrary",),
        ),
    )(pooled, W1, b1.reshape(1, HIDDEN), W2, b2.reshape(1, VOCAB))


def kernel(inputs, emb, W1, b1, W2, b2):
    idx_chunks = inputs.astype(jnp.int32).reshape(32, _NCH, 128)
    pooled = _sc_pool(emb, idx_chunks)
    return _tc_mlp(pooled, W1, b1, W2, b2)
